# Initial kernel scaffold; baseline (speedup 1.0000x reference)
import jax
import jax.numpy as jnp
from jax.experimental import pallas as pl


def kernel(x, pe_table):
    raise NotImplementedError("write your pallas kernel here")


# speedup vs baseline: 1.6664x; 1.6664x over previous
import jax
import jax.numpy as jnp
from jax.experimental import pallas as pl


_BS = 1024


def _add_body(x_ref, pe_ref, o_ref):
    o_ref[...] = x_ref[...] + pe_ref[...]


def kernel(x, pe_table):
    B, S, D = x.shape
    bs = _BS if S % _BS == 0 else S
    grid = (S // bs, B)
    return pl.pallas_call(
        _add_body,
        grid=grid,
        in_specs=[
            pl.BlockSpec((1, bs, D), lambda s, b: (b, s, 0)),
            pl.BlockSpec((bs, D), lambda s, b: (s, 0)),
        ],
        out_specs=pl.BlockSpec((1, bs, D), lambda s, b: (b, s, 0)),
        out_shape=jax.ShapeDtypeStruct((B, S, D), x.dtype),
    )(x, pe_table[:S])


# speedup vs baseline: 1.7356x; 1.0416x over previous
import jax
import jax.numpy as jnp
from jax.experimental import pallas as pl


_BS = 2048


def _add_body(x_ref, pe_ref, o_ref):
    o_ref[...] = x_ref[...] + pe_ref[...]


def kernel(x, pe_table):
    B, S, D = x.shape
    bs = _BS if S % _BS == 0 else S
    grid = (S // bs, B)
    return pl.pallas_call(
        _add_body,
        grid=grid,
        in_specs=[
            pl.BlockSpec((1, bs, D), lambda s, b: (b, s, 0)),
            pl.BlockSpec((bs, D), lambda s, b: (s, 0)),
        ],
        out_specs=pl.BlockSpec((1, bs, D), lambda s, b: (b, s, 0)),
        out_shape=jax.ShapeDtypeStruct((B, S, D), x.dtype),
    )(x, pe_table[:S])
